# baseline (speedup 1.0000x reference)
import functools

import jax
import jax.numpy as jnp
from jax import lax
from jax.experimental import pallas as pl
from jax.experimental.pallas import tpu as pltpu
from jax.experimental.pallas import tpu_sc as plsc

_N = 10000
_NP = 10240
_G = 512
_K = 30
_H = 128
_E = 320000
_EP = 327680
_CHUNKS = 80


_G0 = 2
_G1 = 8


def _segsum_body(deg_pass, x_hbm, src_hbm, dst_hbm, zacc_hbm, ones_hbm,
                 out_hbm, src_idx, dst_idx, rows0, rows1, acc, sem0, sem1):
    c = lax.axis_index("c")
    s = lax.axis_index("s")
    if deg_pass:
        base = (c * 16 + s) * _CHUNKS
        ngroups = _CHUNKS // 16
    else:
        base = jnp.where(c == 0, s * 16 * _G0, 256 * _G0 + s * 16 * _G1)
        ngroups = jnp.where(c == 0, _G0, _G1)
    if deg_pass:
        pltpu.sync_copy(ones_hbm, rows0)
    pltpu.sync_copy(zacc_hbm, acc.at[pl.ds(s * 640, 640)])
    plsc.subcore_barrier()

    if deg_pass:
        def dgroup(g, carry):
            pltpu.sync_copy(dst_hbm.at[pl.ds(base + g * 16, 16)], dst_idx)

            def chunk(j, carry2):
                pltpu.sync_copy(rows0, acc.at[dst_idx.at[j]], add=True)
                return carry2

            lax.fori_loop(0, 16, chunk, carry)
            return carry

        lax.fori_loop(0, _CHUNKS // 16, dgroup, 0)
    else:
        def group(g, carry):
            pltpu.sync_copy(src_hbm.at[pl.ds(base + g * 16, 16)], src_idx)
            pltpu.sync_copy(dst_hbm.at[pl.ds(base + g * 16, 16)], dst_idx)
            pltpu.async_copy(x_hbm.at[src_idx.at[0]], rows0, sem0)

            def chunk2(jj, carry2):
                j = 2 * jj
                pltpu.async_copy(x_hbm.at[src_idx.at[j + 1]], rows1, sem1)
                pltpu.make_async_copy(x_hbm.at[src_idx.at[j]], rows0,
                                      sem0).wait()
                pltpu.sync_copy(rows0, acc.at[dst_idx.at[j]], add=True)

                @pl.when(jj < 7)
                def _():
                    pltpu.async_copy(x_hbm.at[src_idx.at[j + 2]], rows0, sem0)

                pltpu.make_async_copy(x_hbm.at[src_idx.at[j + 1]], rows1,
                                      sem1).wait()
                pltpu.sync_copy(rows1, acc.at[dst_idx.at[j + 1]], add=True)
                return carry2

            lax.fori_loop(0, 8, chunk2, carry)
            return carry

        lax.fori_loop(0, ngroups, group, 0)
    plsc.subcore_barrier()
    pltpu.sync_copy(acc.at[pl.ds(s * 640, 640)],
                    out_hbm.at[c, pl.ds(s * 640, 640)])


def _make_segsum(deg_pass):
    mesh = plsc.VectorSubcoreMesh(core_axis_name="c", subcore_axis_name="s")
    return functools.partial(
        pl.kernel,
        functools.partial(_segsum_body, deg_pass),
        out_type=jax.ShapeDtypeStruct((2, _NP, _H), jnp.float32),
        mesh=mesh,
        scratch_types=[
            pltpu.VMEM((16, 128), jnp.int32),
            pltpu.VMEM((16, 128), jnp.int32),
            pltpu.VMEM((128, _H), jnp.float32),
            pltpu.VMEM((128, _H), jnp.float32),
            pltpu.VMEM_SHARED((_NP, _H), jnp.float32),
            pltpu.SemaphoreType.DMA,
            pltpu.SemaphoreType.DMA,
        ],
    )()


_segsum_deg = _make_segsum(True)
_segsum = _make_segsum(False)


def _sage_body(p0_ref, p1_ref, d0_ref, d1_ref, x_ref, wl_ref, bl_ref, wr_ref,
               out_ref):
    agg = p0_ref[:] + p1_ref[:]
    deg = d0_ref[:, :1] + d1_ref[:, :1]
    rdeg = 1.0 / jnp.maximum(deg, 1.0)
    h = (jnp.dot(agg * rdeg, wl_ref[:], preferred_element_type=jnp.float32)
         + bl_ref[:]
         + jnp.dot(x_ref[:], wr_ref[:], preferred_element_type=jnp.float32))
    out_ref[...] = jnp.maximum(h, 0.0)


def _sage_tc(p0, p1, d0, d1, x, WlT, bl, WrT):
    return pl.pallas_call(
        _sage_body,
        out_shape=jax.ShapeDtypeStruct((_NP, _H), jnp.float32),
    )(p0, p1, d0, d1, x, WlT, bl, WrT)


def _tail_body(pooled_ref, w0_ref, w1_ref, w2_ref, cb_ref, l1r_ref, l1b_ref,
               w2p_ref, b2p_ref, out_ref):
    acc = jnp.zeros((_G, _H), jnp.float32)
    for t in range(28):
        x0 = pooled_ref[:, t, :]
        x1 = pooled_ref[:, t + 1, :]
        x2 = pooled_ref[:, t + 2, :]
        c = (jnp.dot(x0, w0_ref[:], preferred_element_type=jnp.float32)
             + jnp.dot(x1, w1_ref[:], preferred_element_type=jnp.float32)
             + jnp.dot(x2, w2_ref[:], preferred_element_type=jnp.float32)
             + cb_ref[:])
        c = jnp.maximum(c, 0.0)
        acc = acc + jnp.dot(c, l1r_ref[t], preferred_element_type=jnp.float32)
    y = jnp.maximum(acc + l1b_ref[:], 0.0)
    z = jnp.dot(y, w2p_ref[:], preferred_element_type=jnp.float32) + b2p_ref[:]
    mask = lax.broadcasted_iota(jnp.int32, (_G, _H), 1) < 10
    zm = jnp.where(mask, z, -jnp.inf)
    m = jnp.max(zm, axis=1, keepdims=True)
    lse = jnp.log(jnp.sum(jnp.where(mask, jnp.exp(z - m), 0.0),
                          axis=1, keepdims=True)) + m
    out_ref[...] = z - lse


def _tail_tc(pooled, w0, w1, w2, cb, l1r, l1b, w2p, b2p):
    return pl.pallas_call(
        _tail_body,
        out_shape=jax.ShapeDtypeStruct((_G, _H), jnp.float32),
    )(pooled, w0, w1, w2, cb, l1r, l1b, w2p, b2p)


def _sort_pool_xla(x, batch, k):
    n, h = x.shape
    perm = jnp.lexsort((-x[:, -1], batch))
    sx = x[perm]
    sb = batch[perm]
    counts = jnp.bincount(batch, length=_G)
    starts = jnp.concatenate([jnp.zeros((1,), counts.dtype),
                              jnp.cumsum(counts)[:-1]])
    pos = jnp.arange(n) - starts[sb]
    valid = pos < k
    flat_idx = jnp.where(valid, sb * k + pos, _G * _K)
    flat = jnp.zeros((_G * _K + 1, h), x.dtype).at[flat_idx].set(sx)
    return flat[:_G * _K].reshape(_G, _K, h)


def kernel(x, edge_index, batch, k, Wl1, b1, Wr1, Wl2, b2, Wr2, Wl3, b3, Wr3,
           conv1d_w, conv1d_b, lin1_w, lin1_b, lin2_w, lin2_b):
    src = jnp.concatenate(
        [edge_index[0], jnp.full((_EP - _E,), _N, jnp.int32)]).reshape(-1, 128)
    dst = jnp.concatenate(
        [edge_index[1], jnp.full((_EP - _E,), _N, jnp.int32)]).reshape(-1, 128)
    xp = jnp.zeros((_NP, _H), jnp.float32).at[:_N].set(x)
    zacc = jnp.zeros((640, _H), jnp.float32)
    ones = jnp.ones((128, _H), jnp.float32)

    dp = _segsum_deg(xp, src, dst, zacc, ones)
    d0, d1 = dp[0, :, :1], dp[1, :, :1]

    h = xp
    for WlT, bl, WrT in ((Wl1.T, b1, Wr1.T), (Wl2.T, b2, Wr2.T),
                         (Wl3.T, b3, Wr3.T)):
        p = _segsum(h, src, dst, zacc, ones)
        h = _sage_tc(p[0], p[1], d0, d1, h, WlT, bl[None, :], WrT)

    pooled = _sort_pool_xla(h[:_N], batch, k)

    w0 = conv1d_w[:, :, 0].T
    w1 = conv1d_w[:, :, 1].T
    w2 = conv1d_w[:, :, 2].T
    cb = conv1d_b[None, :]
    l1r = lin1_w.reshape(_H, 32, 28).transpose(2, 1, 0)
    l1b = lin1_b[None, :]
    w2p = jnp.zeros((_H, _H), jnp.float32).at[:, :10].set(lin2_w.T)
    b2p = jnp.zeros((1, _H), jnp.float32).at[:, :10].set(lin2_b)

    out = _tail_tc(pooled, w0, w1, w2, cb, l1r, l1b, w2p, b2p)
    return out[:, :10]


# speedup vs baseline: 1.1284x; 1.1284x over previous
import functools

import jax
import jax.numpy as jnp
from jax import lax
from jax.experimental import pallas as pl
from jax.experimental.pallas import tpu as pltpu
from jax.experimental.pallas import tpu_sc as plsc

_N = 10000
_NP = 10240
_G = 512
_K = 30
_H = 128
_E = 320000
_EP = 327680
_CHUNKS = 80


_G0 = 8
_G1 = 2


def _segsum_body(deg_pass, x_hbm, src_hbm, dst_hbm, zacc_hbm, ones_hbm,
                 out_hbm, src_idx, dst_idx, rows0, rows1, acc, sem0, sem1):
    c = lax.axis_index("c")
    s = lax.axis_index("s")
    if deg_pass:
        base = (c * 16 + s) * _CHUNKS
        ngroups = _CHUNKS // 16
    else:
        base = jnp.where(c == 0, s * 16 * _G0, 256 * _G0 + s * 16 * _G1)
        ngroups = jnp.where(c == 0, _G0, _G1)
    if deg_pass:
        pltpu.sync_copy(ones_hbm, rows0)
    pltpu.sync_copy(zacc_hbm, acc.at[pl.ds(s * 640, 640)])
    plsc.subcore_barrier()

    if deg_pass:
        def dgroup(g, carry):
            pltpu.sync_copy(dst_hbm.at[pl.ds(base + g * 16, 16)], dst_idx)

            def chunk(j, carry2):
                pltpu.sync_copy(rows0, acc.at[dst_idx.at[j]], add=True)
                return carry2

            lax.fori_loop(0, 16, chunk, carry)
            return carry

        lax.fori_loop(0, _CHUNKS // 16, dgroup, 0)
    else:
        def group(g, carry):
            pltpu.sync_copy(src_hbm.at[pl.ds(base + g * 16, 16)], src_idx)
            pltpu.sync_copy(dst_hbm.at[pl.ds(base + g * 16, 16)], dst_idx)
            pltpu.async_copy(x_hbm.at[src_idx.at[0]], rows0, sem0)

            def chunk2(jj, carry2):
                j = 2 * jj
                pltpu.async_copy(x_hbm.at[src_idx.at[j + 1]], rows1, sem1)
                pltpu.make_async_copy(x_hbm.at[src_idx.at[j]], rows0,
                                      sem0).wait()
                pltpu.sync_copy(rows0, acc.at[dst_idx.at[j]], add=True)

                @pl.when(jj < 7)
                def _():
                    pltpu.async_copy(x_hbm.at[src_idx.at[j + 2]], rows0, sem0)

                pltpu.make_async_copy(x_hbm.at[src_idx.at[j + 1]], rows1,
                                      sem1).wait()
                pltpu.sync_copy(rows1, acc.at[dst_idx.at[j + 1]], add=True)
                return carry2

            lax.fori_loop(0, 8, chunk2, carry)
            return carry

        lax.fori_loop(0, ngroups, group, 0)
    plsc.subcore_barrier()
    pltpu.sync_copy(acc.at[pl.ds(s * 640, 640)],
                    out_hbm.at[c, pl.ds(s * 640, 640)])


def _make_segsum(deg_pass):
    mesh = plsc.VectorSubcoreMesh(core_axis_name="c", subcore_axis_name="s")
    return functools.partial(
        pl.kernel,
        functools.partial(_segsum_body, deg_pass),
        out_type=jax.ShapeDtypeStruct((2, _NP, _H), jnp.float32),
        mesh=mesh,
        scratch_types=[
            pltpu.VMEM((16, 128), jnp.int32),
            pltpu.VMEM((16, 128), jnp.int32),
            pltpu.VMEM((128, _H), jnp.float32),
            pltpu.VMEM((128, _H), jnp.float32),
            pltpu.VMEM_SHARED((_NP, _H), jnp.float32),
            pltpu.SemaphoreType.DMA,
            pltpu.SemaphoreType.DMA,
        ],
    )()


_segsum_deg = _make_segsum(True)
_segsum = _make_segsum(False)


def _sage_body(p0_ref, p1_ref, d0_ref, d1_ref, x_ref, wl_ref, bl_ref, wr_ref,
               out_ref):
    agg = p0_ref[:] + p1_ref[:]
    deg = d0_ref[:, :1] + d1_ref[:, :1]
    rdeg = 1.0 / jnp.maximum(deg, 1.0)
    h = (jnp.dot(agg * rdeg, wl_ref[:], preferred_element_type=jnp.float32)
         + bl_ref[:]
         + jnp.dot(x_ref[:], wr_ref[:], preferred_element_type=jnp.float32))
    out_ref[...] = jnp.maximum(h, 0.0)


def _sage_tc(p0, p1, d0, d1, x, WlT, bl, WrT):
    return pl.pallas_call(
        _sage_body,
        out_shape=jax.ShapeDtypeStruct((_NP, _H), jnp.float32),
    )(p0, p1, d0, d1, x, WlT, bl, WrT)


def _tail_body(pooled_ref, w0_ref, w1_ref, w2_ref, cb_ref, l1r_ref, l1b_ref,
               w2p_ref, b2p_ref, out_ref):
    acc = jnp.zeros((_G, _H), jnp.float32)
    for t in range(28):
        x0 = pooled_ref[:, t, :]
        x1 = pooled_ref[:, t + 1, :]
        x2 = pooled_ref[:, t + 2, :]
        c = (jnp.dot(x0, w0_ref[:], preferred_element_type=jnp.float32)
             + jnp.dot(x1, w1_ref[:], preferred_element_type=jnp.float32)
             + jnp.dot(x2, w2_ref[:], preferred_element_type=jnp.float32)
             + cb_ref[:])
        c = jnp.maximum(c, 0.0)
        acc = acc + jnp.dot(c, l1r_ref[t], preferred_element_type=jnp.float32)
    y = jnp.maximum(acc + l1b_ref[:], 0.0)
    z = jnp.dot(y, w2p_ref[:], preferred_element_type=jnp.float32) + b2p_ref[:]
    mask = lax.broadcasted_iota(jnp.int32, (_G, _H), 1) < 10
    zm = jnp.where(mask, z, -jnp.inf)
    m = jnp.max(zm, axis=1, keepdims=True)
    lse = jnp.log(jnp.sum(jnp.where(mask, jnp.exp(z - m), 0.0),
                          axis=1, keepdims=True)) + m
    out_ref[...] = z - lse


def _tail_tc(pooled, w0, w1, w2, cb, l1r, l1b, w2p, b2p):
    return pl.pallas_call(
        _tail_body,
        out_shape=jax.ShapeDtypeStruct((_G, _H), jnp.float32),
    )(pooled, w0, w1, w2, cb, l1r, l1b, w2p, b2p)


def _sort_pool_xla(x, batch, k):
    n, h = x.shape
    perm = jnp.lexsort((-x[:, -1], batch))
    sx = x[perm]
    sb = batch[perm]
    counts = jnp.bincount(batch, length=_G)
    starts = jnp.concatenate([jnp.zeros((1,), counts.dtype),
                              jnp.cumsum(counts)[:-1]])
    pos = jnp.arange(n) - starts[sb]
    valid = pos < k
    flat_idx = jnp.where(valid, sb * k + pos, _G * _K)
    flat = jnp.zeros((_G * _K + 1, h), x.dtype).at[flat_idx].set(sx)
    return flat[:_G * _K].reshape(_G, _K, h)


def kernel(x, edge_index, batch, k, Wl1, b1, Wr1, Wl2, b2, Wr2, Wl3, b3, Wr3,
           conv1d_w, conv1d_b, lin1_w, lin1_b, lin2_w, lin2_b):
    src = jnp.concatenate(
        [edge_index[0], jnp.full((_EP - _E,), _N, jnp.int32)]).reshape(-1, 128)
    dst = jnp.concatenate(
        [edge_index[1], jnp.full((_EP - _E,), _N, jnp.int32)]).reshape(-1, 128)
    xp = jnp.zeros((_NP, _H), jnp.float32).at[:_N].set(x)
    zacc = jnp.zeros((640, _H), jnp.float32)
    ones = jnp.ones((128, _H), jnp.float32)

    dp = _segsum_deg(xp, src, dst, zacc, ones)
    d0, d1 = dp[0, :, :1], dp[1, :, :1]

    h = xp
    for WlT, bl, WrT in ((Wl1.T, b1, Wr1.T), (Wl2.T, b2, Wr2.T),
                         (Wl3.T, b3, Wr3.T)):
        p = _segsum(h, src, dst, zacc, ones)
        h = _sage_tc(p[0], p[1], d0, d1, h, WlT, bl[None, :], WrT)

    pooled = _sort_pool_xla(h[:_N], batch, k)

    w0 = conv1d_w[:, :, 0].T
    w1 = conv1d_w[:, :, 1].T
    w2 = conv1d_w[:, :, 2].T
    cb = conv1d_b[None, :]
    l1r = lin1_w.reshape(_H, 32, 28).transpose(2, 1, 0)
    l1b = lin1_b[None, :]
    w2p = jnp.zeros((_H, _H), jnp.float32).at[:, :10].set(lin2_w.T)
    b2p = jnp.zeros((1, _H), jnp.float32).at[:, :10].set(lin2_b)

    out = _tail_tc(pooled, w0, w1, w2, cb, l1r, l1b, w2p, b2p)
    return out[:, :10]
